# baseline (speedup 1.0000x reference)
import functools

import jax
import jax.numpy as jnp
from jax import lax
from jax.experimental import pallas as pl
from jax.experimental.pallas import tpu as pltpu
from jax.experimental.pallas import tpu_sc as plsc

N = 10000
E = 320000
D = 128
ED = 16

NC = 2
NS = 16
NW = NC * NS
E_PER_W = E // NW
CHUNK = 40
NCHUNK = E_PER_W // CHUNK
NPAD = 10240
ROWS_PER_TILE = NPAD // NS


def _eproj_body(a_ref, w_ref, b_ref, o_ref):
    o_ref[...] = (
        lax.dot_general(a_ref[...], w_ref[...], (((0,), (0,)), ((), ())),
                        preferred_element_type=jnp.float32)
        + b_ref[...]
    )


def _edge_proj(edge_attr_t, We, be):
    RB = 2560
    return pl.pallas_call(
        _eproj_body,
        grid=(E // RB,),
        in_specs=[
            pl.BlockSpec((ED, RB), lambda i: (0, i)),
            pl.BlockSpec((ED, D), lambda i: (0, 0)),
            pl.BlockSpec((1, D), lambda i: (0, 0)),
        ],
        out_specs=pl.BlockSpec((RB, D), lambda i: (i, 0)),
        out_shape=jax.ShapeDtypeStruct((E, D), jnp.float32),
    )(edge_attr_t, We, be.reshape(1, D))


def _depad_body(ei_ref, s_ref, d_ref):
    s_ref[...] = ei_ref[0].reshape(s_ref.shape)
    d_ref[...] = ei_ref[1].reshape(d_ref.shape)


def _split_edge_index(edge_index):
    s, d = pl.pallas_call(
        _depad_body,
        grid=(1,),
        in_specs=[pl.BlockSpec((2, E), lambda i: (0, 0))],
        out_specs=[pl.BlockSpec((E // 128, 128), lambda i: (0, 0)),
                   pl.BlockSpec((E // 128, 128), lambda i: (0, 0))],
        out_shape=[jax.ShapeDtypeStruct((E // 128, 128), jnp.int32),
                   jax.ShapeDtypeStruct((E // 128, 128), jnp.int32)],
    )(edge_index)
    return s.reshape(E), d.reshape(E)


def _node_body(x_ref, a_ref, w_ref, b_ref, o_ref):
    t = x_ref[...] + a_ref[0] + a_ref[1]
    y = jnp.dot(t, w_ref[...], preferred_element_type=jnp.float32) + b_ref[...]
    o_ref[...] = jnp.where(y >= 0.0, y, 0.01 * y)


def _node_update(x, agg2, W, b):
    RB = 1000
    return pl.pallas_call(
        _node_body,
        grid=(N // RB,),
        in_specs=[
            pl.BlockSpec((RB, D), lambda i: (i, 0)),
            pl.BlockSpec((NC, RB, D), lambda i: (0, i, 0)),
            pl.BlockSpec((D, D), lambda i: (0, 0)),
            pl.BlockSpec((1, D), lambda i: (0, 0)),
        ],
        out_specs=pl.BlockSpec((RB, D), lambda i: (i, 0)),
        out_shape=jax.ShapeDtypeStruct((N, D), jnp.float32),
    )(x, agg2, W, b.reshape(1, D))


def _gine_sc_body(x_hbm, src_hbm, dst_hbm, e_hbm, out_hbm,
                  srcv, dstv, rows, ev, agg,
                  isem0, isem1, dsem0, dsem1, ssem0, ssem1):
    c = lax.axis_index("c")
    s = lax.axis_index("s")
    w = c * NS + s
    isems = (isem0, isem1)
    dsems = (dsem0, dsem1)
    ssems = (ssem0, ssem1)
    base_w = w * E_PER_W

    def _zrow(r, carry):
        for g in range(D // 16):
            rows[0, r, pl.ds(g * 16, 16)] = jnp.zeros((16,), jnp.float32)
        return carry

    lax.fori_loop(0, CHUNK, _zrow, 0)
    for j in range(ROWS_PER_TILE // CHUNK):
        pltpu.sync_copy(
            rows.at[0], agg.at[pl.ds(s * ROWS_PER_TILE + j * CHUNK, CHUNK)]
        )
    plsc.subcore_barrier()

    def _issue_idx(j, b):
        base = base_w + j * CHUNK
        pltpu.async_copy(src_hbm.at[pl.ds(base, CHUNK)], srcv.at[b], isems[b])
        pltpu.async_copy(dst_hbm.at[pl.ds(base, CHUNK)], dstv.at[b], isems[b])

    def _wait_idx(j, b):
        base = base_w + j * CHUNK
        pltpu.make_async_copy(
            src_hbm.at[pl.ds(base, CHUNK)], srcv.at[b], isems[b]).wait()
        pltpu.make_async_copy(
            dst_hbm.at[pl.ds(base, CHUNK)], dstv.at[b], isems[b]).wait()

    def _issue_data(j, b):
        pltpu.async_copy(x_hbm.at[srcv.at[b]], rows.at[b], dsems[b])
        pltpu.async_copy(
            e_hbm.at[pl.ds(base_w + j * CHUNK, CHUNK)], ev.at[b], dsems[b]
        )

    def _wait_data(j, b):
        pltpu.make_async_copy(
            x_hbm.at[srcv.at[b]], rows.at[b], dsems[b]).wait()
        pltpu.make_async_copy(
            e_hbm.at[pl.ds(base_w + j * CHUNK, CHUNK)], ev.at[b], dsems[b]
        ).wait()

    def _compute(b):
        @plsc.parallel_loop(0, CHUNK, unroll=4)
        def _row(r):
            for g in range(D // 16):
                sl = pl.ds(g * 16, 16)
                rows[b, r, sl] = jnp.maximum(rows[b, r, sl] + ev[b, r, sl], 0.0)

    def _issue_scatter(b):
        pltpu.async_copy(rows.at[b], agg.at[dstv.at[b]], ssems[b], add=True)

    def _wait_scatter(b):
        pltpu.make_async_copy(rows.at[b], agg.at[dstv.at[b]], ssems[b]).wait()

    _issue_idx(0, 0)
    _issue_idx(1, 1)
    _wait_idx(0, 0)
    _issue_data(0, 0)

    def _step(t, carry):
        for b in range(2):
            j = t * 2 + b
            nb = 1 - b
            _wait_idx(j + 1, nb)

            @pl.when(j >= 1)
            def _():
                _wait_scatter(nb)

            _issue_data(j + 1, nb)
            _wait_data(j, b)
            _compute(b)
            _issue_scatter(b)
            _issue_idx(j + 2, b)
        return carry

    lax.fori_loop(0, (NCHUNK - 2) // 2, _step, 0)
    j = NCHUNK - 2
    _wait_idx(j + 1, 1)
    _wait_scatter(1)
    _issue_data(j + 1, 1)
    _wait_data(j, 0)
    _compute(0)
    _issue_scatter(0)
    _wait_data(j + 1, 1)
    _compute(1)
    _issue_scatter(1)
    _wait_scatter(0)
    _wait_scatter(1)

    plsc.subcore_barrier()
    pltpu.sync_copy(
        agg.at[pl.ds(s * ROWS_PER_TILE, ROWS_PER_TILE)],
        out_hbm.at[c, pl.ds(s * ROWS_PER_TILE, ROWS_PER_TILE)],
    )


def _gine_sc(x, src, dst, e):
    mesh = plsc.VectorSubcoreMesh(
        core_axis_name="c", subcore_axis_name="s",
        num_cores=NC, num_subcores=NS,
    )
    run = pl.kernel(
        _gine_sc_body,
        out_type=jax.ShapeDtypeStruct((NC, NPAD, D), jnp.float32),
        mesh=mesh,
        scratch_types=[
            pltpu.VMEM((2, CHUNK), jnp.int32),
            pltpu.VMEM((2, CHUNK), jnp.int32),
            pltpu.VMEM((2, CHUNK, D), jnp.float32),
            pltpu.VMEM((2, CHUNK, D), jnp.float32),
            pltpu.VMEM_SHARED((NPAD, D), jnp.float32),
            pltpu.SemaphoreType.DMA,
            pltpu.SemaphoreType.DMA,
            pltpu.SemaphoreType.DMA,
            pltpu.SemaphoreType.DMA,
            pltpu.SemaphoreType.DMA,
            pltpu.SemaphoreType.DMA,
        ],
    )
    return run(x, src, dst, e)


def kernel(x, edge_index, edge_attr,
           W0, b0, We0, be0, W1, b1, We1, be1, W2, b2, We2, be2):
    src, dst = _split_edge_index(edge_index)
    params = [(W0, b0, We0, be0), (W1, b1, We1, be1), (W2, b2, We2, be2)]
    ea_t = edge_attr.T
    es = [_edge_proj(ea_t, We, be) for (_, _, We, be) in params]
    h = x
    for (W, b, _, _), e in zip(params, es):
        agg2 = _gine_sc(h, src, dst, e)
        h = _node_update(h, agg2, W, b)
    return h


# speedup vs baseline: 1.0315x; 1.0315x over previous
import functools

import jax
import jax.numpy as jnp
from jax import lax
from jax.experimental import pallas as pl
from jax.experimental.pallas import tpu as pltpu
from jax.experimental.pallas import tpu_sc as plsc

N = 10000
E = 320000
D = 128
ED = 16

NC = 2
NS = 16
NW = NC * NS
E_PER_W = E // NW
CHUNK = 40
NCHUNK = E_PER_W // CHUNK
NPAD = 10240
ROWS_PER_TILE = NPAD // NS


def _eproj_body(a_ref, w_ref, b_ref, o_ref):
    o_ref[...] = (
        lax.dot_general(a_ref[...], w_ref[...], (((0,), (0,)), ((), ())),
                        preferred_element_type=jnp.float32)
        + b_ref[...]
    )


def _edge_proj(edge_attr_t, We, be, row_off=0, nrows=E):
    RB = 2560
    off_b = row_off // RB
    return pl.pallas_call(
        _eproj_body,
        grid=(nrows // RB,),
        in_specs=[
            pl.BlockSpec((ED, RB), lambda i: (0, i + off_b)),
            pl.BlockSpec((ED, D), lambda i: (0, 0)),
            pl.BlockSpec((1, D), lambda i: (0, 0)),
        ],
        out_specs=pl.BlockSpec((RB, D), lambda i: (i, 0)),
        out_shape=jax.ShapeDtypeStruct((nrows, D), jnp.float32),
    )(edge_attr_t, We, be.reshape(1, D))


def _depad_body(ei_ref, s_ref, d_ref):
    s_ref[...] = ei_ref[0].reshape(s_ref.shape)
    d_ref[...] = ei_ref[1].reshape(d_ref.shape)


def _split_edge_index(edge_index):
    s, d = pl.pallas_call(
        _depad_body,
        grid=(1,),
        in_specs=[pl.BlockSpec((2, E), lambda i: (0, 0))],
        out_specs=[pl.BlockSpec((E // 128, 128), lambda i: (0, 0)),
                   pl.BlockSpec((E // 128, 128), lambda i: (0, 0))],
        out_shape=[jax.ShapeDtypeStruct((E // 128, 128), jnp.int32),
                   jax.ShapeDtypeStruct((E // 128, 128), jnp.int32)],
    )(edge_index)
    return s.reshape(E), d.reshape(E)


def _node_body(x_ref, a_ref, w_ref, b_ref, o_ref):
    t = x_ref[...] + a_ref[0] + a_ref[1]
    y = jnp.dot(t, w_ref[...], preferred_element_type=jnp.float32) + b_ref[...]
    o_ref[...] = jnp.where(y >= 0.0, y, 0.01 * y)


def _node_body4(x_ref, a_ref, a2_ref, w_ref, b_ref, o_ref):
    t = x_ref[...] + (a_ref[0] + a_ref[1]) + (a2_ref[0] + a2_ref[1])
    y = jnp.dot(t, w_ref[...], preferred_element_type=jnp.float32) + b_ref[...]
    o_ref[...] = jnp.where(y >= 0.0, y, 0.01 * y)


def _node_update4(x, agg2a, agg2b, W, b):
    RB = 1000
    aspec = pl.BlockSpec((NC, RB, D), lambda i: (0, i, 0))
    return pl.pallas_call(
        _node_body4,
        grid=(N // RB,),
        in_specs=[
            pl.BlockSpec((RB, D), lambda i: (i, 0)),
            aspec, aspec,
            pl.BlockSpec((D, D), lambda i: (0, 0)),
            pl.BlockSpec((1, D), lambda i: (0, 0)),
        ],
        out_specs=pl.BlockSpec((RB, D), lambda i: (i, 0)),
        out_shape=jax.ShapeDtypeStruct((N, D), jnp.float32),
    )(x, agg2a, agg2b, W, b.reshape(1, D))


def _node_update(x, agg2, W, b):
    RB = 1000
    return pl.pallas_call(
        _node_body,
        grid=(N // RB,),
        in_specs=[
            pl.BlockSpec((RB, D), lambda i: (i, 0)),
            pl.BlockSpec((NC, RB, D), lambda i: (0, i, 0)),
            pl.BlockSpec((D, D), lambda i: (0, 0)),
            pl.BlockSpec((1, D), lambda i: (0, 0)),
        ],
        out_specs=pl.BlockSpec((RB, D), lambda i: (i, 0)),
        out_shape=jax.ShapeDtypeStruct((N, D), jnp.float32),
    )(x, agg2, W, b.reshape(1, D))


def _gine_sc_body(x_hbm, src_hbm, dst_hbm, e_hbm, out_hbm,
                  srcv, dstv, rows, ev, agg,
                  isem0, isem1, dsem0, dsem1, ssem0, ssem1,
                  *, e_off, nchunk):
    c = lax.axis_index("c")
    s = lax.axis_index("s")
    w = c * NS + s
    isems = (isem0, isem1)
    dsems = (dsem0, dsem1)
    ssems = (ssem0, ssem1)
    epw = nchunk * CHUNK
    base_w = w * epw

    def _zrow(r, carry):
        for g in range(D // 16):
            rows[0, r, pl.ds(g * 16, 16)] = jnp.zeros((16,), jnp.float32)
        return carry

    lax.fori_loop(0, CHUNK, _zrow, 0)
    for j in range(ROWS_PER_TILE // CHUNK):
        pltpu.sync_copy(
            rows.at[0], agg.at[pl.ds(s * ROWS_PER_TILE + j * CHUNK, CHUNK)]
        )
    plsc.subcore_barrier()

    def _issue_idx(j, b):
        base = e_off + base_w + j * CHUNK
        pltpu.async_copy(src_hbm.at[pl.ds(base, CHUNK)], srcv.at[b], isems[b])
        pltpu.async_copy(dst_hbm.at[pl.ds(base, CHUNK)], dstv.at[b], isems[b])

    def _wait_idx(j, b):
        base = e_off + base_w + j * CHUNK
        pltpu.make_async_copy(
            src_hbm.at[pl.ds(base, CHUNK)], srcv.at[b], isems[b]).wait()
        pltpu.make_async_copy(
            dst_hbm.at[pl.ds(base, CHUNK)], dstv.at[b], isems[b]).wait()

    def _issue_data(j, b):
        pltpu.async_copy(x_hbm.at[srcv.at[b]], rows.at[b], dsems[b])
        pltpu.async_copy(
            e_hbm.at[pl.ds(base_w + j * CHUNK, CHUNK)], ev.at[b], dsems[b]
        )

    def _wait_data(j, b):
        pltpu.make_async_copy(
            x_hbm.at[srcv.at[b]], rows.at[b], dsems[b]).wait()
        pltpu.make_async_copy(
            e_hbm.at[pl.ds(base_w + j * CHUNK, CHUNK)], ev.at[b], dsems[b]
        ).wait()

    def _compute(b):
        @plsc.parallel_loop(0, CHUNK, unroll=4)
        def _row(r):
            for g in range(D // 16):
                sl = pl.ds(g * 16, 16)
                rows[b, r, sl] = jnp.maximum(rows[b, r, sl] + ev[b, r, sl], 0.0)

    def _issue_scatter(b):
        pltpu.async_copy(rows.at[b], agg.at[dstv.at[b]], ssems[b], add=True)

    def _wait_scatter(b):
        pltpu.make_async_copy(rows.at[b], agg.at[dstv.at[b]], ssems[b]).wait()

    _issue_idx(0, 0)
    _issue_idx(1, 1)
    _wait_idx(0, 0)
    _issue_data(0, 0)

    def _step(t, carry):
        for b in range(2):
            j = t * 2 + b
            nb = 1 - b
            _wait_idx(j + 1, nb)

            @pl.when(j >= 1)
            def _():
                _wait_scatter(nb)

            _issue_data(j + 1, nb)
            _wait_data(j, b)
            _compute(b)
            _issue_scatter(b)
            _issue_idx(j + 2, b)
        return carry

    lax.fori_loop(0, (nchunk - 2) // 2, _step, 0)
    j = nchunk - 2
    _wait_idx(j + 1, 1)
    _wait_scatter(1)
    _issue_data(j + 1, 1)
    _wait_data(j, 0)
    _compute(0)
    _issue_scatter(0)
    _wait_data(j + 1, 1)
    _compute(1)
    _issue_scatter(1)
    _wait_scatter(0)
    _wait_scatter(1)

    plsc.subcore_barrier()
    pltpu.sync_copy(
        agg.at[pl.ds(s * ROWS_PER_TILE, ROWS_PER_TILE)],
        out_hbm.at[c, pl.ds(s * ROWS_PER_TILE, ROWS_PER_TILE)],
    )


def _gine_sc(x, src, dst, e, e_off=0, nchunk=NCHUNK):
    mesh = plsc.VectorSubcoreMesh(
        core_axis_name="c", subcore_axis_name="s",
        num_cores=NC, num_subcores=NS,
    )
    run = pl.kernel(
        functools.partial(_gine_sc_body, e_off=e_off, nchunk=nchunk),
        out_type=jax.ShapeDtypeStruct((NC, NPAD, D), jnp.float32),
        mesh=mesh,
        scratch_types=[
            pltpu.VMEM((2, CHUNK), jnp.int32),
            pltpu.VMEM((2, CHUNK), jnp.int32),
            pltpu.VMEM((2, CHUNK, D), jnp.float32),
            pltpu.VMEM((2, CHUNK, D), jnp.float32),
            pltpu.VMEM_SHARED((NPAD, D), jnp.float32),
            pltpu.SemaphoreType.DMA,
            pltpu.SemaphoreType.DMA,
            pltpu.SemaphoreType.DMA,
            pltpu.SemaphoreType.DMA,
            pltpu.SemaphoreType.DMA,
            pltpu.SemaphoreType.DMA,
        ],
    )
    return run(x, src, dst, e)


def kernel(x, edge_index, edge_attr,
           W0, b0, We0, be0, W1, b1, We1, be1, W2, b2, We2, be2):
    src, dst = _split_edge_index(edge_index)
    ea_t = edge_attr.T
    EA = 161280
    e0a = _edge_proj(ea_t, We0, be0, 0, EA)
    agg_a = _gine_sc(x, src, dst, e0a, 0, EA // NW // CHUNK)
    e0b = _edge_proj(ea_t, We0, be0, EA, E - EA)
    e1 = _edge_proj(ea_t, We1, be1)
    e2 = _edge_proj(ea_t, We2, be2)
    agg_b = _gine_sc(x, src, dst, e0b, EA, (E - EA) // NW // CHUNK)
    h = _node_update4(x, agg_a, agg_b, W0, b0)
    agg2 = _gine_sc(h, src, dst, e1)
    h = _node_update(h, agg2, W1, b1)
    agg2 = _gine_sc(h, src, dst, e2)
    h = _node_update(h, agg2, W2, b2)
    return h
